# baseline (speedup 1.0000x reference)
import functools

import jax
import jax.numpy as jnp
from jax import lax
from jax.experimental import pallas as pl
from jax.experimental.pallas import tpu as pltpu
from jax.experimental.pallas import tpu_sc as plsc

_B, _C, _H, _W = 16, 768, 32, 32
_HALF = 384
_HW = _H * _W
_NS = 16
_CPS = _C // _NS
_BPC = _B // 2


def _pos_body(row_hbm, col_hbm, out_hbm, tab_v, chunk_v, shared, sem):
    cid = lax.axis_index("c")
    sid = lax.axis_index("s")
    c0 = sid * _CPS
    is_row_half = sid < (_NS // 2)

    @pl.when(is_row_half)
    def _():
        pltpu.sync_copy(row_hbm.at[pl.ds(0, _H * _HALF)], tab_v)

        def body_c(cp, carry):
            base = cp * _HW
            c = c0 + cp
            for h in range(_H):
                idx = jnp.full((16,), h * _HALF, jnp.int32) + c
                v = plsc.load_gather(tab_v, [idx])
                chunk_v[pl.ds(base + h * _W, 16)] = v
                chunk_v[pl.ds(base + h * _W + 16, 16)] = v
            return carry

        lax.fori_loop(0, _CPS, body_c, 0)

    @pl.when(jnp.logical_not(is_row_half))
    def _():
        pltpu.sync_copy(col_hbm.at[pl.ds(0, _W * _HALF)], tab_v)

        def body_c(cp, carry):
            base = cp * _HW
            c = (c0 - _HALF) + cp
            i16 = lax.iota(jnp.int32, 16)
            vlo = plsc.load_gather(tab_v, [i16 * _HALF + c])
            vhi = plsc.load_gather(tab_v, [(i16 + 16) * _HALF + c])
            for h in range(_H):
                chunk_v[pl.ds(base + h * _W, 16)] = vlo
                chunk_v[pl.ds(base + h * _W + 16, 16)] = vhi
            return carry

        lax.fori_loop(0, _CPS, body_c, 0)

    n = _CPS * _HW
    pltpu.sync_copy(chunk_v, shared.at[pl.ds(c0 * _HW, n)])
    copies = [
        pltpu.async_copy(
            shared.at[pl.ds(c0 * _HW, n)],
            out_hbm.at[cid * _BPC + k, pl.ds(c0 * _HW, n)],
            sem,
        )
        for k in range(_BPC)
    ]
    for c in copies:
        c.wait()


@jax.jit
def _pos_encode(row_embed, col_embed):
    mesh = plsc.VectorSubcoreMesh(core_axis_name="c", subcore_axis_name="s")
    run = functools.partial(
        pl.kernel,
        out_type=jax.ShapeDtypeStruct((_B, _C * _HW), jnp.float32),
        mesh=mesh,
        compiler_params=pltpu.CompilerParams(needs_layout_passes=False),
        scratch_types=[
            pltpu.VMEM((_H * _HALF,), jnp.float32),
            pltpu.VMEM((_CPS * _HW,), jnp.float32),
            pltpu.VMEM_SHARED((_C * _HW,), jnp.float32),
            pltpu.SemaphoreType.DMA,
        ],
    )(_pos_body)
    flat = run(row_embed.reshape(-1), col_embed.reshape(-1))
    return flat.reshape(_B, _C, _H, _W)


def kernel(feat, row_embed, col_embed):
    del feat
    return _pos_encode(row_embed, col_embed)


# speedup vs baseline: 3.3951x; 3.3951x over previous
import functools

import jax
import jax.numpy as jnp
from jax import lax
from jax.experimental import pallas as pl
from jax.experimental.pallas import tpu as pltpu
from jax.experimental.pallas import tpu_sc as plsc

_B, _C, _H, _W = 16, 768, 32, 32
_HALF = 384
_HW = _H * _W
_NS = 16
_CPS = _C // _NS
_BPC = _B // 2


def _pos_body(row_hbm, col_hbm, out_hbm, tab_v, chunk_v, shared, sem):
    cid = lax.axis_index("c")
    sid = lax.axis_index("s")
    c0 = sid * _CPS
    is_row_half = sid < (_NS // 2)

    @pl.when(is_row_half)
    def _():
        pltpu.sync_copy(row_hbm.at[pl.ds(0, _H * _HALF)], tab_v)

        def body_c(cp, carry):
            c = c0 + cp
            for s in range(8):
                for t in range(0, 8, 2):
                    h = (s * 128 + t * 16) // 32
                    idx = jnp.full((16,), h * _HALF, jnp.int32) + c
                    v = plsc.load_gather(tab_v, [idx])
                    chunk_v[cp, s, pl.ds(t * 16, 16)] = v
                    chunk_v[cp, s, pl.ds(t * 16 + 16, 16)] = v
            return carry

        lax.fori_loop(0, _CPS, body_c, 0)

    @pl.when(jnp.logical_not(is_row_half))
    def _():
        pltpu.sync_copy(col_hbm.at[pl.ds(0, _W * _HALF)], tab_v)

        def body_c(cp, carry):
            c = (c0 - _HALF) + cp
            i16 = lax.iota(jnp.int32, 16)
            vlo = plsc.load_gather(tab_v, [i16 * _HALF + c])
            vhi = plsc.load_gather(tab_v, [(i16 + 16) * _HALF + c])
            for s in range(8):
                for t in range(0, 8, 2):
                    chunk_v[cp, s, pl.ds(t * 16, 16)] = vlo
                    chunk_v[cp, s, pl.ds(t * 16 + 16, 16)] = vhi
            return carry

        lax.fori_loop(0, _CPS, body_c, 0)

    pltpu.sync_copy(chunk_v, shared.at[pl.ds(c0, _CPS)])
    copies = [
        pltpu.async_copy(
            shared.at[pl.ds(c0, _CPS)],
            out_hbm.at[cid * _BPC + k, pl.ds(c0, _CPS)],
            sem,
        )
        for k in range(_BPC)
    ]
    for c in copies:
        c.wait()


@jax.jit
def _pos_encode(row_embed, col_embed):
    mesh = plsc.VectorSubcoreMesh(core_axis_name="c", subcore_axis_name="s")
    run = functools.partial(
        pl.kernel,
        out_type=jax.ShapeDtypeStruct((_B, _C, 8, 128), jnp.float32),
        mesh=mesh,
        compiler_params=pltpu.CompilerParams(needs_layout_passes=False),
        scratch_types=[
            pltpu.VMEM((_H * _HALF,), jnp.float32),
            pltpu.VMEM((_CPS, 8, 128), jnp.float32),
            pltpu.VMEM_SHARED((_C, 8, 128), jnp.float32),
            pltpu.SemaphoreType.DMA,
        ],
    )(_pos_body)
    out = run(row_embed.reshape(-1), col_embed.reshape(-1))
    return out.reshape(_B, _C, _H, _W)


def kernel(feat, row_embed, col_embed):
    del feat
    return _pos_encode(row_embed, col_embed)


# speedup vs baseline: 3.4044x; 1.0028x over previous
import functools

import jax
import jax.numpy as jnp
from jax import lax
from jax.experimental import pallas as pl
from jax.experimental.pallas import tpu as pltpu
from jax.experimental.pallas import tpu_sc as plsc

_B, _C, _H, _W = 16, 768, 32, 32
_HALF = 384
_HW = _H * _W
_NS = 16
_CPS = _C // _NS
_BPC = _B // 2


def _pos_body(row_hbm, col_hbm, out_hbm, tab_v, chunk_v, shared, sem):
    cid = lax.axis_index("c")
    sid = lax.axis_index("s")
    c0 = sid * _CPS
    is_row_half = sid < (_NS // 2)

    _dn = lax.GatherDimensionNumbers(
        offset_dims=(), collapsed_slice_dims=(0,), start_index_map=(0,)
    )

    def _splat(vec, lane):
        return lax.gather(
            vec,
            jnp.full((16, 1), lane, jnp.int32),
            _dn,
            slice_sizes=(1,),
            mode=lax.GatherScatterMode.PROMISE_IN_BOUNDS,
        )

    @pl.when(is_row_half)
    def _():
        pltpu.sync_copy(row_hbm.at[pl.ds(0, _H * _HALF)], tab_v)

        def body_c(cp, carry):
            c = c0 + cp
            i16 = lax.iota(jnp.int32, 16)
            vh_lo = plsc.load_gather(tab_v, [i16 * _HALF + c])
            vh_hi = plsc.load_gather(tab_v, [(i16 + 16) * _HALF + c])
            for s in range(8):
                for t in range(0, 8, 2):
                    h = (s * 128 + t * 16) // 32
                    v = _splat(vh_lo if h < 16 else vh_hi, h % 16)
                    chunk_v[cp, s, pl.ds(t * 16, 16)] = v
                    chunk_v[cp, s, pl.ds(t * 16 + 16, 16)] = v
            return carry

        lax.fori_loop(0, _CPS, body_c, 0)

    @pl.when(jnp.logical_not(is_row_half))
    def _():
        pltpu.sync_copy(col_hbm.at[pl.ds(0, _W * _HALF)], tab_v)

        def body_c(cp, carry):
            c = (c0 - _HALF) + cp
            i16 = lax.iota(jnp.int32, 16)
            vlo = plsc.load_gather(tab_v, [i16 * _HALF + c])
            vhi = plsc.load_gather(tab_v, [(i16 + 16) * _HALF + c])
            for s in range(8):
                for t in range(0, 8, 2):
                    chunk_v[cp, s, pl.ds(t * 16, 16)] = vlo
                    chunk_v[cp, s, pl.ds(t * 16 + 16, 16)] = vhi
            return carry

        lax.fori_loop(0, _CPS, body_c, 0)

    pltpu.sync_copy(chunk_v, shared.at[pl.ds(c0, _CPS)])
    copies = [
        pltpu.async_copy(
            shared.at[pl.ds(c0, _CPS)],
            out_hbm.at[cid * _BPC + k, pl.ds(c0, _CPS)],
            sem,
        )
        for k in range(_BPC)
    ]
    for c in copies:
        c.wait()


@jax.jit
def _pos_encode(row_embed, col_embed):
    mesh = plsc.VectorSubcoreMesh(core_axis_name="c", subcore_axis_name="s")
    run = functools.partial(
        pl.kernel,
        out_type=jax.ShapeDtypeStruct((_B, _C, 8, 128), jnp.float32),
        mesh=mesh,
        compiler_params=pltpu.CompilerParams(needs_layout_passes=False),
        scratch_types=[
            pltpu.VMEM((_H * _HALF,), jnp.float32),
            pltpu.VMEM((_CPS, 8, 128), jnp.float32),
            pltpu.VMEM_SHARED((_C, 8, 128), jnp.float32),
            pltpu.SemaphoreType.DMA,
        ],
    )(_pos_body)
    out = run(row_embed.reshape(-1), col_embed.reshape(-1))
    return out.reshape(_B, _C, _H, _W)


def kernel(feat, row_embed, col_embed):
    del feat
    return _pos_encode(row_embed, col_embed)


# speedup vs baseline: 3.4269x; 1.0066x over previous
import functools

import jax
import jax.numpy as jnp
from jax import lax
from jax.experimental import pallas as pl
from jax.experimental.pallas import tpu as pltpu
from jax.experimental.pallas import tpu_sc as plsc

_B, _C, _H, _W = 16, 768, 32, 32
_HALF = 384
_HW = _H * _W
_NS = 16
_CPS = _C // _NS
_BPC = _B // 2


def _pos_body(row_hbm, col_hbm, out_hbm, tab_v, chunk_v, shared, sem):
    cid = lax.axis_index("c")
    sid = lax.axis_index("s")
    c0 = sid * _CPS
    is_row_half = sid < (_NS // 2)

    _dn = lax.GatherDimensionNumbers(
        offset_dims=(), collapsed_slice_dims=(0,), start_index_map=(0,)
    )

    def _splat(vec, lane):
        return lax.gather(
            vec,
            jnp.full((16, 1), lane, jnp.int32),
            _dn,
            slice_sizes=(1,),
            mode=lax.GatherScatterMode.PROMISE_IN_BOUNDS,
        )

    @pl.when(is_row_half)
    def _():
        pltpu.sync_copy(row_hbm.at[pl.ds(0, _H * _HALF)], tab_v)

        def body_c(cp, carry):
            c = c0 + cp
            i16 = lax.iota(jnp.int32, 16)
            vh_lo = plsc.load_gather(tab_v, [i16 * _HALF + c])
            vh_hi = plsc.load_gather(tab_v, [(i16 + 16) * _HALF + c])
            for s in range(8):
                for t in range(0, 8, 2):
                    h = (s * 128 + t * 16) // 32
                    v = _splat(vh_lo if h < 16 else vh_hi, h % 16)
                    chunk_v[cp, s, pl.ds(t * 16, 16)] = v
                    chunk_v[cp, s, pl.ds(t * 16 + 16, 16)] = v
            return carry

        lax.fori_loop(0, _CPS, body_c, 0)

    @pl.when(jnp.logical_not(is_row_half))
    def _():
        pltpu.sync_copy(col_hbm.at[pl.ds(0, _W * _HALF)], tab_v)

        def body_c(cp, carry):
            c = (c0 - _HALF) + cp
            i16 = lax.iota(jnp.int32, 16)
            vlo = plsc.load_gather(tab_v, [i16 * _HALF + c])
            vhi = plsc.load_gather(tab_v, [(i16 + 16) * _HALF + c])
            for s in range(8):
                for t in range(0, 8, 2):
                    chunk_v[cp, s, pl.ds(t * 16, 16)] = vlo
                    chunk_v[cp, s, pl.ds(t * 16 + 16, 16)] = vhi
            return carry

        lax.fori_loop(0, _CPS, body_c, 0)

    pltpu.sync_copy(chunk_v, shared.at[pl.ds(c0, _CPS)])
    plsc.subcore_barrier()

    b = cid * _BPC + (sid // 2)
    h0 = (sid % 2) * (_C // 2)
    pltpu.async_copy(
        shared.at[pl.ds(h0, _C // 2)],
        out_hbm.at[b, pl.ds(h0, _C // 2)],
        sem,
    ).wait()


@jax.jit
def _pos_encode(row_embed, col_embed):
    mesh = plsc.VectorSubcoreMesh(core_axis_name="c", subcore_axis_name="s")
    run = functools.partial(
        pl.kernel,
        out_type=jax.ShapeDtypeStruct((_B, _C, 8, 128), jnp.float32),
        mesh=mesh,
        compiler_params=pltpu.CompilerParams(needs_layout_passes=False),
        scratch_types=[
            pltpu.VMEM((_H * _HALF,), jnp.float32),
            pltpu.VMEM((_CPS, 8, 128), jnp.float32),
            pltpu.VMEM_SHARED((_C, 8, 128), jnp.float32),
            pltpu.SemaphoreType.DMA,
        ],
    )(_pos_body)
    out = run(row_embed.reshape(-1), col_embed.reshape(-1))
    return out.reshape(_B, _C, _H, _W)


def kernel(feat, row_embed, col_embed):
    del feat
    return _pos_encode(row_embed, col_embed)


# speedup vs baseline: 6.4628x; 1.8859x over previous
import functools

import jax
import jax.numpy as jnp
from jax import lax
from jax.experimental import pallas as pl
from jax.experimental.pallas import tpu as pltpu
from jax.experimental.pallas import tpu_sc as plsc

_B, _C, _H, _W = 16, 768, 32, 32
_HALF = 384
_NS = 16
_HPS = _H // _NS
_BPC = _B // 2
_TPB = _H * 4 * 6
_ROWS_PER_HALF = _TPB // 2


def _pos_body(row_hbm, col_hbm, out_hbm, rowbuf_v, rowtiles_v, image, sem):
    cid = lax.axis_index("c")
    sid = lax.axis_index("s")

    copies = []
    for i in range(_HPS):
        h = sid * _HPS + i
        pltpu.sync_copy(row_hbm.at[pl.ds(h * _HALF, _HALF)], rowbuf_v.at[i])
        for ct in range(3):
            for v in range(8):
                vec = rowbuf_v[i, pl.ds(ct * 128 + v * 16, 16)]
                for r in range(8):
                    rowtiles_v[i, ct, r, pl.ds(v * 16, 16)] = vec
        for wt in range(4):
            n0 = (h * 4 + wt) * 6
            copies.append(
                pltpu.async_copy(rowtiles_v.at[i], image.at[pl.ds(n0, 3)], sem)
            )
            copies.append(
                pltpu.async_copy(col_hbm.at[wt], image.at[pl.ds(n0 + 3, 3)], sem)
            )
    for c in copies:
        c.wait()
    plsc.subcore_barrier()

    b = cid * _BPC + (sid // 2)
    r0 = (sid % 2) * _ROWS_PER_HALF
    pltpu.async_copy(
        image.at[pl.ds(r0, _ROWS_PER_HALF)],
        out_hbm.at[pl.ds(b * _TPB + r0, _ROWS_PER_HALF)],
        sem,
    ).wait()


@jax.jit
def _pos_encode(row_embed, col_embed):
    mesh = plsc.VectorSubcoreMesh(core_axis_name="c", subcore_axis_name="s")
    run = functools.partial(
        pl.kernel,
        out_type=jax.ShapeDtypeStruct((_B * _TPB, 8, 128), jnp.float32),
        mesh=mesh,
        compiler_params=pltpu.CompilerParams(needs_layout_passes=False),
        scratch_types=[
            pltpu.VMEM((_HPS, _HALF), jnp.float32),
            pltpu.VMEM((_HPS, 3, 8, 128), jnp.float32),
            pltpu.VMEM_SHARED((_TPB, 8, 128), jnp.float32),
            pltpu.SemaphoreType.DMA,
        ],
    )(_pos_body)
    col6 = (
        col_embed[:_W].reshape(4, 8, 3, 128).transpose(0, 2, 1, 3)
    )
    out = run(row_embed.reshape(-1), col6)
    out = out.reshape(_B, _H, 4, 6, 8, 128).transpose(0, 3, 5, 1, 2, 4)
    return out.reshape(_B, _C, _H, _W)


def kernel(feat, row_embed, col_embed):
    del feat
    return _pos_encode(row_embed, col_embed)
